# Initial kernel scaffold; baseline (speedup 1.0000x reference)
import jax
import jax.numpy as jnp
from jax.experimental import pallas as pl


def kernel(x, edge_index, rel_type, weight):
    raise NotImplementedError("write your pallas kernel here")


# speedup vs baseline: 20.9872x; 20.9872x over previous
import functools

import jax
import jax.numpy as jnp
from jax import lax
from jax.experimental import pallas as pl
from jax.experimental.pallas import tpu as pltpu
from jax.experimental.pallas import tpu_sc as plsc

N = 10000
E = 320000
F = 128
R = 8

NC = 2
NS = 16
NW = NC * NS
EPW = E // NW
CHUNK = 80
NCHUNK = EPW // CHUNK
NWOUT = 10
ROWS_PT = N // NWOUT
DEGW = 16
ZROWS = 40
DZROWS = 200


_BN = 1000


def _xw_body(x_ref, w_ref, o_ref):
    o_ref[0] = jnp.dot(x_ref[...], w_ref[0], preferred_element_type=jnp.float32)


def _compute_xw(x, weight):
    return pl.pallas_call(
        _xw_body,
        grid=(N // _BN, R),
        in_specs=[
            pl.BlockSpec((_BN, F), lambda i, r: (i, 0)),
            pl.BlockSpec((1, F, F), lambda i, r: (r, 0, 0)),
        ],
        out_specs=pl.BlockSpec((1, _BN, F), lambda i, r: (r, i, 0)),
        out_shape=jax.ShapeDtypeStruct((R, N, F), jnp.float32),
    )(x, weight)


def _sc_body(xw_hbm, src_hbm, dst_hbm, rel_hbm, out_hbm, deg_hbm,
             sd_v, gidx_v, rows_v, ones_v, dz_v, acc_s, deg_s, sem):
    c = lax.axis_index("c")
    s = lax.axis_index("s")
    wid = s * NC + c

    z16 = jnp.zeros((16,), jnp.float32)

    def _zrow(j, _):
        for k in range(F // 16):
            rows_v[j, pl.ds(k * 16, 16)] = z16
        ones_v[j, :] = jnp.ones((16,), jnp.float32)
        return 0
    lax.fori_loop(0, CHUNK, _zrow, 0)

    def _dzrow(j, _):
        dz_v[j, :] = z16
        return 0
    lax.fori_loop(0, DZROWS, _dzrow, 0)

    @pl.when(s < NWOUT)
    def _init():
        for z in range(ROWS_PT // ZROWS):
            pltpu.sync_copy(
                rows_v.at[pl.ds(0, ZROWS)],
                acc_s.at[pl.ds(s * ROWS_PT + z * ZROWS, ZROWS)])
        for z in range(ROWS_PT // DZROWS):
            pltpu.sync_copy(
                dz_v, deg_s.at[pl.ds(s * ROWS_PT + z * DZROWS, DZROWS)])

    pltpu.sync_copy(rel_hbm.at[wid], gidx_v)
    pltpu.sync_copy(src_hbm.at[wid], sd_v)

    def _gidx(j, _):
        for k in range(CHUNK // 16):
            sl = pl.ds(k * 16, 16)
            gidx_v[j, sl] = gidx_v[j, sl] * N + sd_v[j, sl]
        return 0
    lax.fori_loop(0, NCHUNK, _gidx, 0)

    pltpu.sync_copy(dst_hbm.at[wid], sd_v)

    plsc.subcore_barrier()

    def _step(j, _):
        pltpu.async_copy(xw_hbm.at[gidx_v.at[j]], rows_v, sem).wait()
        pltpu.sync_copy(rows_v, acc_s.at[sd_v.at[j]], add=True)
        pltpu.sync_copy(ones_v, deg_s.at[sd_v.at[j]], add=True)
        return 0
    lax.fori_loop(0, NCHUNK, _step, 0)

    plsc.subcore_barrier()

    @pl.when(s < NWOUT)
    def _writeout():
        sl = pl.ds(s * ROWS_PT, ROWS_PT)
        pltpu.sync_copy(acc_s.at[sl], out_hbm.at[c, sl])
        pltpu.sync_copy(deg_s.at[sl], deg_hbm.at[c, sl])


def _sc_scatter(xw_flat, src3, dst3, rel3):
    mesh = plsc.VectorSubcoreMesh(core_axis_name="c", subcore_axis_name="s")
    f = functools.partial(
        pl.kernel,
        out_type=[
            jax.ShapeDtypeStruct((NC, N, F), jnp.float32),
            jax.ShapeDtypeStruct((NC, N, DEGW), jnp.float32),
        ],
        mesh=mesh,
        compiler_params=pltpu.CompilerParams(use_tc_tiling_on_sc=False),
        scratch_types=[
            pltpu.VMEM((NCHUNK, CHUNK), jnp.int32),
            pltpu.VMEM((NCHUNK, CHUNK), jnp.int32),
            pltpu.VMEM((CHUNK, F), jnp.float32),
            pltpu.VMEM((CHUNK, DEGW), jnp.float32),
            pltpu.VMEM((DZROWS, DEGW), jnp.float32),
            pltpu.VMEM_SHARED((N, F), jnp.float32),
            pltpu.VMEM_SHARED((N, DEGW), jnp.float32),
            pltpu.SemaphoreType.DMA,
        ],
    )(_sc_body)
    return f(xw_flat, src3, dst3, rel3)


def _fin_body(p_ref, d_ref, o_ref):
    psum = p_ref[0] + p_ref[1]
    deg = d_ref[0, :, 0:1] + d_ref[1, :, 0:1]
    o_ref[...] = psum / jnp.maximum(deg, 1.0)


def _finalize(partials, degs):
    return pl.pallas_call(
        _fin_body,
        grid=(N // _BN,),
        in_specs=[
            pl.BlockSpec((NC, _BN, F), lambda i: (0, i, 0)),
            pl.BlockSpec((NC, _BN, DEGW), lambda i: (0, i, 0)),
        ],
        out_specs=pl.BlockSpec((_BN, F), lambda i: (i, 0)),
        out_shape=jax.ShapeDtypeStruct((N, F), jnp.float32),
    )(partials, degs)


def kernel(x, edge_index, rel_type, weight):
    src = edge_index[0].astype(jnp.int32).reshape(NW, NCHUNK, CHUNK)
    dst = edge_index[1].astype(jnp.int32).reshape(NW, NCHUNK, CHUNK)
    rel = rel_type.astype(jnp.int32).reshape(NW, NCHUNK, CHUNK)
    xw = _compute_xw(x, weight).reshape(R * N, F)
    partials, degs = _sc_scatter(xw, src, dst, rel)
    return _finalize(partials, degs)


# speedup vs baseline: 24.9141x; 1.1871x over previous
import functools

import jax
import jax.numpy as jnp
from jax import lax
from jax.experimental import pallas as pl
from jax.experimental.pallas import tpu as pltpu
from jax.experimental.pallas import tpu_sc as plsc

N = 10000
E = 320000
F = 128
R = 8

NC = 2
NS = 16
NW = NC * NS
EPW = E // NW
CHUNK = 40
NCHUNK = EPW // CHUNK
NWOUT = 10
ROWS_PT = N // NWOUT
DEGW = 16
ZROWS = CHUNK
DZROWS = 50


_BN = 1000


def _xw_body(x_ref, w_ref, o_ref):
    o_ref[0] = jnp.dot(x_ref[...], w_ref[0], preferred_element_type=jnp.float32)


def _compute_xw(x, weight):
    return pl.pallas_call(
        _xw_body,
        grid=(N // _BN, R),
        in_specs=[
            pl.BlockSpec((_BN, F), lambda i, r: (i, 0)),
            pl.BlockSpec((1, F, F), lambda i, r: (r, 0, 0)),
        ],
        out_specs=pl.BlockSpec((1, _BN, F), lambda i, r: (r, i, 0)),
        out_shape=jax.ShapeDtypeStruct((R, N, F), jnp.float32),
    )(x, weight)


def _sc_body(xw_hbm, gidx_hbm, dst_hbm, out_hbm, deg_hbm,
             gidx_v, dst_v, rows_v, ones_v, dz_v,
             acc_s, deg_s, gsem, dsem):
    c = lax.axis_index("c")
    s = lax.axis_index("s")
    wid = s * NC + c

    z16 = jnp.zeros((16,), jnp.float32)

    def _zrow(j, _):
        for k in range(F // 16):
            rows_v[0, j, pl.ds(k * 16, 16)] = z16
        return 0
    lax.fori_loop(0, ZROWS, _zrow, 0)

    def _orow(j, _):
        ones_v[j, :] = jnp.ones((16,), jnp.float32)
        return 0
    lax.fori_loop(0, CHUNK, _orow, 0)

    def _dzrow(j, _):
        dz_v[j, :] = z16
        return 0
    lax.fori_loop(0, DZROWS, _dzrow, 0)

    @pl.when(s < NWOUT)
    def _init():
        for z in range(ROWS_PT // ZROWS):
            pltpu.sync_copy(
                rows_v.at[0, pl.ds(0, ZROWS)],
                acc_s.at[pl.ds(s * ROWS_PT + z * ZROWS, ZROWS)])
        for z in range(ROWS_PT // DZROWS):
            pltpu.sync_copy(
                dz_v, deg_s.at[pl.ds(s * ROWS_PT + z * DZROWS, DZROWS)])

    pltpu.sync_copy(gidx_hbm.at[wid], gidx_v)
    pltpu.sync_copy(dst_hbm.at[wid], dst_v)

    pltpu.async_copy(xw_hbm.at[gidx_v.at[0]], rows_v.at[0], gsem)
    pltpu.async_copy(xw_hbm.at[gidx_v.at[1]], rows_v.at[1], gsem)

    plsc.subcore_barrier()

    @pl.loop(0, NCHUNK, step=2)
    def _step(g):
        for b in range(2):
            j = g + b
            pltpu.make_async_copy(
                xw_hbm.at[gidx_v.at[j]], rows_v.at[b], gsem).wait()
            pltpu.sync_copy(ones_v, deg_s.at[dst_v.at[j]], add=True)
            pltpu.sync_copy(rows_v.at[b], acc_s.at[dst_v.at[j]], add=True)

            @pl.when(j + 2 < NCHUNK)
            def _next_gather():
                pltpu.async_copy(
                    xw_hbm.at[gidx_v.at[j + 2]], rows_v.at[b], gsem)

    plsc.subcore_barrier()

    @pl.when(s < NWOUT)
    def _writeout():
        sl = pl.ds(s * ROWS_PT, ROWS_PT)
        pltpu.sync_copy(acc_s.at[sl], out_hbm.at[c, sl])
        pltpu.sync_copy(deg_s.at[sl], deg_hbm.at[c, sl])


def _sc_scatter(xw_flat, gidx3, dst3):
    mesh = plsc.VectorSubcoreMesh(core_axis_name="c", subcore_axis_name="s")
    f = functools.partial(
        pl.kernel,
        out_type=[
            jax.ShapeDtypeStruct((NC, N, F), jnp.float32),
            jax.ShapeDtypeStruct((NC, N, DEGW), jnp.float32),
        ],
        mesh=mesh,
        compiler_params=pltpu.CompilerParams(use_tc_tiling_on_sc=False),
        scratch_types=[
            pltpu.VMEM((NCHUNK, CHUNK), jnp.int32),
            pltpu.VMEM((NCHUNK, CHUNK), jnp.int32),
            pltpu.VMEM((2, CHUNK, F), jnp.float32),
            pltpu.VMEM((CHUNK, DEGW), jnp.float32),
            pltpu.VMEM((DZROWS, DEGW), jnp.float32),
            pltpu.VMEM_SHARED((N, F), jnp.float32),
            pltpu.VMEM_SHARED((N, DEGW), jnp.float32),
            pltpu.SemaphoreType.DMA,
            pltpu.SemaphoreType.DMA,
        ],
    )(_sc_body)
    return f(xw_flat, gidx3, dst3)


def _fin_body(p_ref, d_ref, o_ref):
    psum = p_ref[0] + p_ref[1]
    deg = d_ref[0, :, 0:1] + d_ref[1, :, 0:1]
    o_ref[...] = psum / jnp.maximum(deg, 1.0)


def _finalize(partials, degs):
    return pl.pallas_call(
        _fin_body,
        grid=(N // _BN,),
        in_specs=[
            pl.BlockSpec((NC, _BN, F), lambda i: (0, i, 0)),
            pl.BlockSpec((NC, _BN, DEGW), lambda i: (0, i, 0)),
        ],
        out_specs=pl.BlockSpec((_BN, F), lambda i: (i, 0)),
        out_shape=jax.ShapeDtypeStruct((N, F), jnp.float32),
    )(partials, degs)


def kernel(x, edge_index, rel_type, weight):
    src = edge_index[0].astype(jnp.int32)
    dst = edge_index[1].astype(jnp.int32)
    rel = rel_type.astype(jnp.int32)
    gidx = (rel * N + src).reshape(NW, NCHUNK, CHUNK)
    dst3 = dst.reshape(NW, NCHUNK, CHUNK)
    xw = _compute_xw(x, weight).reshape(R * N, F)
    partials, degs = _sc_scatter(xw, gidx, dst3)
    return _finalize(partials, degs)


# speedup vs baseline: 25.5044x; 1.0237x over previous
import functools

import jax
import jax.numpy as jnp
from jax import lax
from jax.experimental import pallas as pl
from jax.experimental.pallas import tpu as pltpu
from jax.experimental.pallas import tpu_sc as plsc

N = 10000
E = 320000
F = 128
R = 8

NC = 2
NS = 16
NW = NC * NS
EPW = E // NW
CHUNK = 40
NCHUNK = EPW // CHUNK
NWOUT = 10
ROWS_PT = N // NWOUT
DEGW = 16
ZROWS = CHUNK
DZROWS = 50


_BN = 1000


def _xw_body(x_ref, w_ref, o_ref):
    o_ref[0] = jnp.dot(x_ref[...], w_ref[0], preferred_element_type=jnp.float32)


def _compute_xw(x, weight):
    return pl.pallas_call(
        _xw_body,
        grid=(N // _BN, R),
        in_specs=[
            pl.BlockSpec((_BN, F), lambda i, r: (i, 0)),
            pl.BlockSpec((1, F, F), lambda i, r: (r, 0, 0)),
        ],
        out_specs=pl.BlockSpec((1, _BN, F), lambda i, r: (r, i, 0)),
        out_shape=jax.ShapeDtypeStruct((R, N, F), jnp.float32),
    )(x, weight)


def _sc_body(xw_hbm, gidx_hbm, dst_hbm, out_hbm, deg_hbm,
             gidx_v, dst_v, rows_v, ones_v, dz_v,
             acc_s, deg_s, gsem, dsem):
    c = lax.axis_index("c")
    s = lax.axis_index("s")
    wid = s * NC + c

    z16 = jnp.zeros((16,), jnp.float32)

    def _zrow(j, _):
        for k in range(F // 16):
            rows_v[0, j, pl.ds(k * 16, 16)] = z16
        return 0
    lax.fori_loop(0, ZROWS, _zrow, 0)

    def _orow(j, _):
        ones_v[j, :] = jnp.ones((16,), jnp.float32)
        return 0
    lax.fori_loop(0, CHUNK, _orow, 0)

    def _dzrow(j, _):
        dz_v[j, :] = z16
        return 0
    lax.fori_loop(0, DZROWS, _dzrow, 0)

    @pl.when(s < NWOUT)
    def _init():
        for z in range(ROWS_PT // ZROWS):
            pltpu.sync_copy(
                rows_v.at[0, pl.ds(0, ZROWS)],
                acc_s.at[pl.ds(s * ROWS_PT + z * ZROWS, ZROWS)])
        for z in range(ROWS_PT // DZROWS):
            pltpu.sync_copy(
                dz_v, deg_s.at[pl.ds(s * ROWS_PT + z * DZROWS, DZROWS)])

    pltpu.sync_copy(gidx_hbm.at[wid], gidx_v)
    pltpu.sync_copy(dst_hbm.at[wid], dst_v)

    pltpu.async_copy(xw_hbm.at[gidx_v.at[0]], rows_v.at[0], gsem)
    pltpu.async_copy(xw_hbm.at[gidx_v.at[1]], rows_v.at[1], gsem)

    plsc.subcore_barrier()

    @pl.loop(0, NCHUNK, step=2)
    def _step(g):
        for b in range(2):
            j = g + b
            pltpu.make_async_copy(
                xw_hbm.at[gidx_v.at[j]], rows_v.at[b], gsem).wait()
            pltpu.async_copy(ones_v, deg_s.at[dst_v.at[j]], dsem)
            pltpu.sync_copy(rows_v.at[b], acc_s.at[dst_v.at[j]], add=True)

            @pl.when(j + 2 < NCHUNK)
            def _next_gather():
                pltpu.async_copy(
                    xw_hbm.at[gidx_v.at[j + 2]], rows_v.at[b], gsem)

    def _drain(j, _):
        pltpu.make_async_copy(ones_v, deg_s.at[dst_v.at[j]], dsem).wait()
        return 0
    lax.fori_loop(0, NCHUNK, _drain, 0)

    plsc.subcore_barrier()

    @pl.when(s < NWOUT)
    def _writeout():
        sl = pl.ds(s * ROWS_PT, ROWS_PT)
        pltpu.sync_copy(acc_s.at[sl], out_hbm.at[c, sl])
        pltpu.sync_copy(deg_s.at[sl], deg_hbm.at[c, sl])


def _sc_scatter(xw_flat, gidx3, dst3):
    mesh = plsc.VectorSubcoreMesh(core_axis_name="c", subcore_axis_name="s")
    f = functools.partial(
        pl.kernel,
        out_type=[
            jax.ShapeDtypeStruct((NC, N, F), jnp.float32),
            jax.ShapeDtypeStruct((NC, N, DEGW), jnp.float32),
        ],
        mesh=mesh,
        compiler_params=pltpu.CompilerParams(use_tc_tiling_on_sc=False),
        scratch_types=[
            pltpu.VMEM((NCHUNK, CHUNK), jnp.int32),
            pltpu.VMEM((NCHUNK, CHUNK), jnp.int32),
            pltpu.VMEM((2, CHUNK, F), jnp.float32),
            pltpu.VMEM((CHUNK, DEGW), jnp.float32),
            pltpu.VMEM((DZROWS, DEGW), jnp.float32),
            pltpu.VMEM_SHARED((N, F), jnp.float32),
            pltpu.VMEM_SHARED((N, DEGW), jnp.float32),
            pltpu.SemaphoreType.DMA,
            pltpu.SemaphoreType.DMA,
        ],
    )(_sc_body)
    return f(xw_flat, gidx3, dst3)


def _fin_body(p_ref, d_ref, o_ref):
    psum = p_ref[0] + p_ref[1]
    deg = d_ref[0, :, 0:1] + d_ref[1, :, 0:1]
    o_ref[...] = psum / jnp.maximum(deg, 1.0)


def _finalize(partials, degs):
    return pl.pallas_call(
        _fin_body,
        grid=(N // _BN,),
        in_specs=[
            pl.BlockSpec((NC, _BN, F), lambda i: (0, i, 0)),
            pl.BlockSpec((NC, _BN, DEGW), lambda i: (0, i, 0)),
        ],
        out_specs=pl.BlockSpec((_BN, F), lambda i: (i, 0)),
        out_shape=jax.ShapeDtypeStruct((N, F), jnp.float32),
    )(partials, degs)


def kernel(x, edge_index, rel_type, weight):
    src = edge_index[0].astype(jnp.int32)
    dst = edge_index[1].astype(jnp.int32)
    rel = rel_type.astype(jnp.int32)
    gidx = (rel * N + src).reshape(NW, NCHUNK, CHUNK)
    dst3 = dst.reshape(NW, NCHUNK, CHUNK)
    xw = _compute_xw(x, weight).reshape(R * N, F)
    partials, degs = _sc_scatter(xw, gidx, dst3)
    return _finalize(partials, degs)
